# baseline (speedup 1.0000x reference)
import functools
import jax
import jax.numpy as jnp
from jax import lax
from jax.experimental import pallas as pl
from jax.experimental.pallas import tpu as pltpu
from jax.experimental.pallas import tpu_sc as plsc

_N = 10000
_E = 320000
_HID = 16
_DOUT = 128

_NC = 2
_NS = 16
_NW = _NC * _NS
_CH = 125
_CHUNKS = 80
_NBUF = 8
_OUTER = _CHUNKS // _NBUF
_TPW = _CHUNKS * _CH
_ZR = 640
_NA = _ZR * _NS

_mesh = plsc.VectorSubcoreMesh(core_axis_name="c", subcore_axis_name="s")


@functools.partial(
    pl.kernel,
    out_type=jax.ShapeDtypeStruct((_NC, _NA), jnp.float32),
    mesh=_mesh,
    scratch_types=[
        pltpu.VMEM_SHARED((_NA,), jnp.float32),
        pltpu.VMEM((_CHUNKS, _CH), jnp.int32),
        pltpu.VMEM((128,), jnp.float32),
        pltpu.VMEM((_ZR,), jnp.float32),
    ],
    compiler_params=pltpu.CompilerParams(use_tc_tiling_on_sc=False),
)
def _deg_sc(eidx_hbm, out_hbm, acc, cidx2d, ones_v, zbuf):
    c = lax.axis_index("c")
    s = lax.axis_index("s")
    wid = s * _NC + c

    one16 = jnp.full((16,), 1.0, jnp.float32)
    zero16 = jnp.zeros((16,), jnp.float32)
    for i in range(8):
        ones_v[pl.ds(i * 16, 16)] = one16

    @pl.loop(0, _ZR // 16)
    def _zero(i):
        zbuf[pl.ds(i * 16, 16)] = zero16

    pltpu.sync_copy(zbuf, acc.at[pl.ds(s * _ZR, _ZR)])
    pltpu.sync_copy(eidx_hbm.at[1, wid], cidx2d)
    plsc.subcore_barrier()

    @pl.loop(0, _CHUNKS)
    def _edges(i):
        pltpu.sync_copy(ones_v.at[pl.ds(0, _CH)], acc.at[cidx2d.at[i]],
                        add=True)

    plsc.subcore_barrier()
    pltpu.sync_copy(acc.at[pl.ds(s * _ZR, _ZR)], out_hbm.at[c, pl.ds(s * _ZR, _ZR)])


@functools.partial(
    pl.kernel,
    out_type=jax.ShapeDtypeStruct((_NC, _NA, _HID), jnp.float32),
    mesh=_mesh,
    scratch_types=[
        pltpu.VMEM_SHARED((_NA, _HID), jnp.float32),
        pltpu.VMEM_SHARED((_NA, _HID), jnp.float32),
        pltpu.VMEM((_CHUNKS, _CH), jnp.int32),
        pltpu.VMEM((_CHUNKS, _CH), jnp.int32),
        pltpu.VMEM((2 * _NBUF, _CH, _HID), jnp.float32),
        pltpu.VMEM((_ZR, _HID), jnp.float32),
        [pltpu.SemaphoreType.DMA] * _NBUF,
        [pltpu.SemaphoreType.DMA] * _NBUF,
    ],
    compiler_params=pltpu.CompilerParams(use_tc_tiling_on_sc=False),
)
def _prop_sc(h_hbm, eidx_hbm, out_hbm, acc, h_s, ridx2d, cidx2d,
             rows, zbuf, gsems, ssems):
    c = lax.axis_index("c")
    s = lax.axis_index("s")
    wid = s * _NC + c

    zero16 = jnp.zeros((16,), jnp.float32)

    @pl.loop(0, _ZR)
    def _zero(i):
        zbuf[i, :] = zero16

    pltpu.sync_copy(zbuf, acc.at[pl.ds(s * _ZR, _ZR)])
    nrow = _N // _NS
    pltpu.sync_copy(h_hbm.at[pl.ds(s * nrow, nrow)],
                    h_s.at[pl.ds(s * nrow, nrow)])
    pltpu.sync_copy(eidx_hbm.at[0, wid], ridx2d)
    pltpu.sync_copy(eidx_hbm.at[1, wid], cidx2d)
    plsc.subcore_barrier()

    for b in range(_NBUF):
        pltpu.async_copy(h_s.at[ridx2d.at[b]], rows.at[b], gsems[b])

    @pl.loop(0, _OUTER)
    def _outer(t):
        for b in range(_NBUF):
            i = t * _NBUF + b
            rb = jnp.where(t % 2 == 0, b, b + _NBUF)
            rb_next = jnp.where(t % 2 == 0, b + _NBUF, b)
            pltpu.make_async_copy(h_s.at[ridx2d.at[i]], rows.at[rb],
                                  gsems[b]).wait()
            pltpu.async_copy(rows.at[rb], acc.at[cidx2d.at[i]],
                             ssems[b], add=True)

            @pl.when(t < _OUTER - 1)
            def _next():
                @pl.when(t >= 1)
                def _reclaim():
                    pltpu.make_async_copy(
                        rows.at[rb_next], acc.at[cidx2d.at[i]],
                        ssems[b]).wait()

                pltpu.async_copy(h_s.at[ridx2d.at[i + _NBUF]],
                                 rows.at[rb_next], gsems[b])

    for b in range(_NBUF):
        pltpu.make_async_copy(rows.at[b],
                              acc.at[cidx2d.at[_CHUNKS - 2 * _NBUF + b]],
                              ssems[b]).wait()
        pltpu.make_async_copy(rows.at[b],
                              acc.at[cidx2d.at[_CHUNKS - _NBUF + b]],
                              ssems[b]).wait()

    plsc.subcore_barrier()
    pltpu.sync_copy(acc.at[pl.ds(s * _ZR, _ZR)],
                    out_hbm.at[c, pl.ds(s * _ZR, _ZR)])


def _sigmoid(v):
    return 1.0 / (1.0 + jnp.exp(-v))


def _dg(a, b, ca, cb):
    return lax.dot_general(a, b, (((ca,), (cb,)), ((), ())),
                           preferred_element_type=jnp.float32)


def _tc1_body(x_ref, w1_ref, degp_ref, dis_ref, xw1_ref, ht1_ref):
    xw1t = _dg(w1_ref[...], x_ref[...], 0, 1)
    deg = degp_ref[0:1, 0:_N] + degp_ref[1:2, 0:_N] + 1.0
    dis = lax.rsqrt(deg)
    dis_ref[...] = dis
    xw1_ref[...] = xw1t
    ht1_ref[...] = dis * xw1t


def _psum(p_ref):
    return p_ref[:, 0:_N]


def _tc2_body(p1_ref, dis_ref, xw1_ref, w2_ref, b1_ref,
              x1_ref, xw2_ref, ht2_ref):
    dis = dis_ref[...]
    x1t = jnp.maximum(dis * _psum(p1_ref) + (dis * dis) * xw1_ref[...]
                      + b1_ref[...], 0.0)
    xw2t = _dg(w2_ref[...], x1t, 0, 0)
    x1_ref[...] = x1t
    xw2_ref[...] = xw2t
    ht2_ref[...] = dis * xw2t


def _lstm_cell(xt, h, c, wi, wh, b):
    g = _dg(wi, xt, 1, 0) + b
    if h is not None:
        g = g + _dg(wh, h, 1, 0)
    sg = _sigmoid(g)
    tg = jnp.tanh(g)
    i = sg[0:32, :]
    f = sg[32:64, :]
    gg = tg[64:96, :]
    o = sg[96:128, :]
    c = (i * gg) if c is None else (f * c + i * gg)
    return o * jnp.tanh(c), c


def _tc3_body(p2_ref, dis_ref, xw2_ref, b2_ref, x1_ref,
              wif_ref, whf_ref, bif_ref, bhf_ref,
              wir_ref, whr_ref, bir_ref, bhr_ref,
              watt_ref, batt_ref, xj_ref, ht3_ref):
    dis = dis_ref[...]
    x1t = x1_ref[...]
    x2t = jnp.maximum(dis * _psum(p2_ref) + (dis * dis) * xw2_ref[...]
                      + b2_ref[...], 0.0)

    wif = wif_ref[...]
    whf = whf_ref[...]
    bf = bif_ref[...] + bhf_ref[...]
    wir = wir_ref[...]
    whr = whr_ref[...]
    br = bir_ref[...] + bhr_ref[...]

    hf1, cf1 = _lstm_cell(x1t, None, None, wif, whf, bf)
    hf2, _ = _lstm_cell(x2t, hf1, cf1, wif, whf, bf)
    hb2, cb2 = _lstm_cell(x2t, None, None, wir, whr, br)
    hb1, _ = _lstm_cell(x1t, hb2, cb2, wir, whr, br)

    watt = watt_ref[...]
    wf = watt[0:32, :]
    wb = watt[32:64, :]
    batt = batt_ref[0, 0]
    a0 = (jnp.sum(wf * hf1, axis=0, keepdims=True)
          + jnp.sum(wb * hb1, axis=0, keepdims=True) + batt)
    a1 = (jnp.sum(wf * hf2, axis=0, keepdims=True)
          + jnp.sum(wb * hb2, axis=0, keepdims=True) + batt)
    m = jnp.maximum(a0, a1)
    e0 = jnp.exp(a0 - m)
    e1 = jnp.exp(a1 - m)
    inv = 1.0 / (e0 + e1)
    xjt = (e0 * inv) * x1t + (e1 * inv) * x2t
    xj_ref[...] = xjt
    ht3_ref[...] = dis * xjt


def _tc4_body(p3_ref, dis_ref, xj_ref, wl_ref, bl_ref, out_ref):
    dis = dis_ref[...]
    zt = dis * _psum(p3_ref) + (dis * dis) * xj_ref[...]
    z = _dg(zt, wl_ref[...], 0, 0) + bl_ref[...]
    m = jnp.max(z, axis=1, keepdims=True)
    ez = jnp.exp(z - m)
    lse = jnp.log(jnp.sum(ez, axis=1, keepdims=True)) + m
    out_ref[...] = z - lse


def _f32(shape):
    return jax.ShapeDtypeStruct(shape, jnp.float32)


_tc1 = pl.pallas_call(
    _tc1_body,
    out_shape=[_f32((1, _N)), _f32((_HID, _N)), _f32((_HID, _N))])
_tc2 = pl.pallas_call(
    _tc2_body,
    out_shape=[_f32((_HID, _N)), _f32((_HID, _N)), _f32((_HID, _N))])
_tc3 = pl.pallas_call(
    _tc3_body,
    out_shape=[_f32((_HID, _N)), _f32((_HID, _N))])
_tc4 = pl.pallas_call(_tc4_body, out_shape=_f32((_N, _DOUT)))


def kernel(x, edge_index, W1, b1, W2, b2, W_ih_f, W_hh_f, b_ih_f, b_hh_f,
           W_ih_r, W_hh_r, b_ih_r, b_hh_r, W_att, b_att, Wl, bl):
    eidx = edge_index.astype(jnp.int32).reshape(2, _NW, _CHUNKS, _CH)

    degp = _deg_sc(eidx)

    dis, xw1t, ht1t = _tc1(x, W1, degp)

    def _merge(p):
        return (p[0] + p[1]).T

    p1 = _merge(_prop_sc(ht1t.T, eidx))
    x1t, xw2t, ht2t = _tc2(p1, dis, xw1t, W2, b1.reshape(_HID, 1))

    p2 = _merge(_prop_sc(ht2t.T, eidx))
    xjt, ht3t = _tc3(
        p2, dis, xw2t, b2.reshape(_HID, 1), x1t,
        W_ih_f, W_hh_f, b_ih_f.reshape(128, 1), b_hh_f.reshape(128, 1),
        W_ih_r, W_hh_r, b_ih_r.reshape(128, 1), b_hh_r.reshape(128, 1),
        W_att, b_att.reshape(1, 1))

    p3 = _merge(_prop_sc(ht3t.T, eidx))
    return _tc4(p3, dis, xjt, Wl, bl.reshape(1, _DOUT))


# speedup vs baseline: 1.1232x; 1.1232x over previous
import functools
import jax
import jax.numpy as jnp
from jax import lax
from jax.experimental import pallas as pl
from jax.experimental.pallas import tpu as pltpu
from jax.experimental.pallas import tpu_sc as plsc

_N = 10000
_E = 320000
_HID = 16
_DOUT = 128

_NC = 2
_NS = 16
_NW = _NC * _NS
_CH = 125
_CHUNKS = 80
_NBUF = 8
_OUTER = _CHUNKS // _NBUF
_TPW = _CHUNKS * _CH
_ZR = 640
_NA = _ZR * _NS

_mesh = plsc.VectorSubcoreMesh(core_axis_name="c", subcore_axis_name="s")


@functools.partial(
    pl.kernel,
    out_type=jax.ShapeDtypeStruct((_NC, _NA), jnp.float32),
    mesh=_mesh,
    scratch_types=[
        pltpu.VMEM_SHARED((_NA,), jnp.float32),
        pltpu.VMEM((_CHUNKS, _CH), jnp.int32),
        pltpu.VMEM((128,), jnp.float32),
        pltpu.VMEM((_ZR,), jnp.float32),
    ],
    compiler_params=pltpu.CompilerParams(use_tc_tiling_on_sc=False),
)
def _deg_sc(eidx_hbm, out_hbm, acc, cidx2d, ones_v, zbuf):
    c = lax.axis_index("c")
    s = lax.axis_index("s")
    wid = s * _NC + c

    one16 = jnp.full((16,), 1.0, jnp.float32)
    zero16 = jnp.zeros((16,), jnp.float32)
    for i in range(8):
        ones_v[pl.ds(i * 16, 16)] = one16

    @pl.loop(0, _ZR // 16)
    def _zero(i):
        zbuf[pl.ds(i * 16, 16)] = zero16

    pltpu.sync_copy(zbuf, acc.at[pl.ds(s * _ZR, _ZR)])
    pltpu.sync_copy(eidx_hbm.at[1, wid], cidx2d)
    plsc.subcore_barrier()

    @pl.loop(0, _CHUNKS)
    def _edges(i):
        pltpu.sync_copy(ones_v.at[pl.ds(0, _CH)], acc.at[cidx2d.at[i]],
                        add=True)

    plsc.subcore_barrier()
    pltpu.sync_copy(acc.at[pl.ds(s * _ZR, _ZR)], out_hbm.at[c, pl.ds(s * _ZR, _ZR)])


@functools.partial(
    pl.kernel,
    out_type=jax.ShapeDtypeStruct((_NC, _NA, _HID), jnp.float32),
    mesh=_mesh,
    scratch_types=[
        pltpu.VMEM_SHARED((_NA, _HID), jnp.float32),
        pltpu.VMEM_SHARED((_NA, _HID), jnp.float32),
        pltpu.VMEM((_CHUNKS, _CH), jnp.int32),
        pltpu.VMEM((_CHUNKS, _CH), jnp.int32),
        pltpu.VMEM((2 * _NBUF, _CH, _HID), jnp.float32),
        pltpu.VMEM((_ZR, _HID), jnp.float32),
        [pltpu.SemaphoreType.DMA] * _NBUF,
        [pltpu.SemaphoreType.DMA] * _NBUF,
    ],
    compiler_params=pltpu.CompilerParams(use_tc_tiling_on_sc=False),
)
def _prop_sc(h_hbm, eidx_hbm, out_hbm, acc, h_s, ridx2d, cidx2d,
             rows, zbuf, gsems, ssems):
    c = lax.axis_index("c")
    s = lax.axis_index("s")
    wid = s * _NC + c

    zero16 = jnp.zeros((16,), jnp.float32)

    @pl.loop(0, _ZR)
    def _zero(i):
        zbuf[i, :] = zero16

    pltpu.sync_copy(zbuf, acc.at[pl.ds(s * _ZR, _ZR)])
    nrow = _N // _NS
    pltpu.sync_copy(h_hbm.at[pl.ds(s * nrow, nrow)],
                    h_s.at[pl.ds(s * nrow, nrow)])
    pltpu.sync_copy(eidx_hbm.at[0, wid], ridx2d)
    pltpu.sync_copy(eidx_hbm.at[1, wid], cidx2d)
    plsc.subcore_barrier()

    for b in range(_NBUF):
        pltpu.async_copy(h_s.at[ridx2d.at[b]], rows.at[b], gsems[b])

    @pl.loop(0, _OUTER)
    def _outer(t):
        for b in range(_NBUF):
            i = t * _NBUF + b
            rb = jnp.where(t % 2 == 0, b, b + _NBUF)
            rb_next = jnp.where(t % 2 == 0, b + _NBUF, b)
            pltpu.make_async_copy(h_s.at[ridx2d.at[i]], rows.at[rb],
                                  gsems[b]).wait()
            pltpu.async_copy(rows.at[rb], acc.at[cidx2d.at[i]],
                             ssems[b], add=True)

            @pl.when(t < _OUTER - 1)
            def _next():
                @pl.when(t >= 1)
                def _reclaim():
                    pltpu.make_async_copy(
                        rows.at[rb_next], acc.at[cidx2d.at[i]],
                        ssems[b]).wait()

                pltpu.async_copy(h_s.at[ridx2d.at[i + _NBUF]],
                                 rows.at[rb_next], gsems[b])

    for b in range(_NBUF):
        pltpu.make_async_copy(rows.at[b],
                              acc.at[cidx2d.at[_CHUNKS - 2 * _NBUF + b]],
                              ssems[b]).wait()
        pltpu.make_async_copy(rows.at[b],
                              acc.at[cidx2d.at[_CHUNKS - _NBUF + b]],
                              ssems[b]).wait()

    plsc.subcore_barrier()
    pltpu.sync_copy(acc.at[pl.ds(s * _ZR, _ZR)],
                    out_hbm.at[c, pl.ds(s * _ZR, _ZR)])


def _sigmoid(v):
    return 1.0 / (1.0 + jnp.exp(-v))


def _dg(a, b, ca, cb):
    return lax.dot_general(a, b, (((ca,), (cb,)), ((), ())),
                           preferred_element_type=jnp.float32)


def _tc1_body(x_ref, w1_ref, degp_ref, dis_ref, xw1_ref, ht1_ref):
    xw1t = _dg(w1_ref[...], x_ref[...], 0, 1)
    deg = degp_ref[0:1, 0:_N] + degp_ref[1:2, 0:_N] + 1.0
    dis = lax.rsqrt(deg)
    dis_ref[...] = dis
    xw1_ref[...] = xw1t
    ht1_ref[...] = dis * xw1t


def _psum(p_ref):
    return p_ref[0, :, 0:_N] + p_ref[1, :, 0:_N]


def _tc2_body(p1_ref, dis_ref, xw1_ref, w2_ref, b1_ref,
              x1_ref, xw2_ref, ht2_ref):
    dis = dis_ref[...]
    x1t = jnp.maximum(dis * _psum(p1_ref) + (dis * dis) * xw1_ref[...]
                      + b1_ref[...], 0.0)
    xw2t = _dg(w2_ref[...], x1t, 0, 0)
    x1_ref[...] = x1t
    xw2_ref[...] = xw2t
    ht2_ref[...] = dis * xw2t


def _lstm_cell(xt, h, c, wi, wh, b):
    g = _dg(wi, xt, 1, 0) + b
    if h is not None:
        g = g + _dg(wh, h, 1, 0)
    sg = _sigmoid(g)
    tg = jnp.tanh(g)
    i = sg[0:32, :]
    f = sg[32:64, :]
    gg = tg[64:96, :]
    o = sg[96:128, :]
    c = (i * gg) if c is None else (f * c + i * gg)
    return o * jnp.tanh(c), c


def _tc3_body(p2_ref, dis_ref, xw2_ref, b2_ref, x1_ref,
              wif_ref, whf_ref, bif_ref, bhf_ref,
              wir_ref, whr_ref, bir_ref, bhr_ref,
              watt_ref, batt_ref, xj_ref, ht3_ref):
    dis = dis_ref[...]
    x1t = x1_ref[...]
    x2t = jnp.maximum(dis * _psum(p2_ref) + (dis * dis) * xw2_ref[...]
                      + b2_ref[...], 0.0)

    wif = wif_ref[...]
    whf = whf_ref[...]
    bf = bif_ref[...] + bhf_ref[...]
    wir = wir_ref[...]
    whr = whr_ref[...]
    br = bir_ref[...] + bhr_ref[...]

    hf1, cf1 = _lstm_cell(x1t, None, None, wif, whf, bf)
    hf2, _ = _lstm_cell(x2t, hf1, cf1, wif, whf, bf)
    hb2, cb2 = _lstm_cell(x2t, None, None, wir, whr, br)
    hb1, _ = _lstm_cell(x1t, hb2, cb2, wir, whr, br)

    watt = watt_ref[...]
    wf = watt[0:32, :]
    wb = watt[32:64, :]
    batt = batt_ref[0, 0]
    a0 = (jnp.sum(wf * hf1, axis=0, keepdims=True)
          + jnp.sum(wb * hb1, axis=0, keepdims=True) + batt)
    a1 = (jnp.sum(wf * hf2, axis=0, keepdims=True)
          + jnp.sum(wb * hb2, axis=0, keepdims=True) + batt)
    m = jnp.maximum(a0, a1)
    e0 = jnp.exp(a0 - m)
    e1 = jnp.exp(a1 - m)
    inv = 1.0 / (e0 + e1)
    xjt = (e0 * inv) * x1t + (e1 * inv) * x2t
    xj_ref[...] = xjt
    ht3_ref[...] = dis * xjt


def _tc4_body(p3_ref, dis_ref, xj_ref, wl_ref, bl_ref, out_ref):
    dis = dis_ref[...]
    zt = dis * _psum(p3_ref) + (dis * dis) * xj_ref[...]
    z = _dg(zt, wl_ref[...], 0, 0) + bl_ref[...]
    m = jnp.max(z, axis=1, keepdims=True)
    ez = jnp.exp(z - m)
    lse = jnp.log(jnp.sum(ez, axis=1, keepdims=True)) + m
    out_ref[...] = z - lse


def _f32(shape):
    return jax.ShapeDtypeStruct(shape, jnp.float32)


_tc1 = pl.pallas_call(
    _tc1_body,
    out_shape=[_f32((1, _N)), _f32((_HID, _N)), _f32((_HID, _N))])
_tc2 = pl.pallas_call(
    _tc2_body,
    out_shape=[_f32((_HID, _N)), _f32((_HID, _N)), _f32((_HID, _N))])
_tc3 = pl.pallas_call(
    _tc3_body,
    out_shape=[_f32((_HID, _N)), _f32((_HID, _N))])
_tc4 = pl.pallas_call(_tc4_body, out_shape=_f32((_N, _DOUT)))


def kernel(x, edge_index, W1, b1, W2, b2, W_ih_f, W_hh_f, b_ih_f, b_hh_f,
           W_ih_r, W_hh_r, b_ih_r, b_hh_r, W_att, b_att, Wl, bl):
    eidx = edge_index.astype(jnp.int32).reshape(2, _NW, _CHUNKS, _CH)

    degp = _deg_sc(eidx)

    dis, xw1t, ht1t = _tc1(x, W1, degp)

    def _merge(p):
        return p.transpose(0, 2, 1)

    p1 = _merge(_prop_sc(ht1t.T, eidx))
    x1t, xw2t, ht2t = _tc2(p1, dis, xw1t, W2, b1.reshape(_HID, 1))

    p2 = _merge(_prop_sc(ht2t.T, eidx))
    xjt, ht3t = _tc3(
        p2, dis, xw2t, b2.reshape(_HID, 1), x1t,
        W_ih_f, W_hh_f, b_ih_f.reshape(128, 1), b_hh_f.reshape(128, 1),
        W_ih_r, W_hh_r, b_ih_r.reshape(128, 1), b_hh_r.reshape(128, 1),
        W_att, b_att.reshape(1, 1))

    p3 = _merge(_prop_sc(ht3t.T, eidx))
    return _tc4(p3, dis, xjt, Wl, bl.reshape(1, _DOUT))
